# baseline (speedup 1.0000x reference)
import jax
import jax.numpy as jnp
from jax import lax
from jax.experimental import pallas as pl
from jax.experimental.pallas import tpu as pltpu
from jax.experimental.pallas import tpu_sc as plsc

VOCAB = 100000
EMBED_DIM = 64
B = 4096
T = 200
N = B * T

NC = 2
NS = 16
NW = NC * NS

PER_B = B // NW
L = 16


def _fire_gather(table_hbm, idx_v, rows, sem, t):
    pltpu.async_copy(table_hbm.at[idx_v.at[t]], rows, sem)


def _wait_rows(table_hbm, rows, sem):
    pltpu.make_async_copy(table_hbm.at[pl.ds(0, PER_B)], rows, sem).wait()


def _fire_out(out_hbm, tbuf, sem, b0, t):
    pltpu.async_copy(tbuf, out_hbm.at[t, :, pl.ds(b0, PER_B)], sem)


def _wait_out(out_hbm, tbuf, sem):
    pltpu.make_async_copy(tbuf, out_hbm.at[0, :, pl.ds(0, PER_B)], sem).wait()


def _transpose(rows, tbuf, iota16):
    def dbody(d, carry):
        cidx = jnp.zeros((L,), jnp.int32) + d
        for jb in range(PER_B // L):
            v = plsc.load_gather(rows, [iota16 + (L * jb), cidx])
            tbuf[d, pl.ds(L * jb, L)] = v
        return carry

    lax.fori_loop(0, EMBED_DIM, dbody, 0)


def _body(idxt_hbm, table_hbm, out_hbm,
          idx_v, rows0, rows1, tb0, tb1, g0, g1, o0, o1):
    wid = lax.axis_index("s") * NC + lax.axis_index("c")
    b0 = wid * PER_B
    rows = (rows0, rows1)
    tbuf = (tb0, tb1)
    gsem = (g0, g1)
    osem = (o0, o1)
    iota16 = lax.iota(jnp.int32, L)

    pltpu.sync_copy(idxt_hbm.at[:, pl.ds(b0, PER_B)], idx_v)

    _fire_gather(table_hbm, idx_v, rows[0], gsem[0], 0)
    _fire_gather(table_hbm, idx_v, rows[1], gsem[1], 1)
    _wait_rows(table_hbm, rows[0], gsem[0])
    _transpose(rows[0], tbuf[0], iota16)
    _fire_out(out_hbm, tbuf[0], osem[0], b0, 0)
    _fire_gather(table_hbm, idx_v, rows[0], gsem[0], 2)
    _wait_rows(table_hbm, rows[1], gsem[1])
    _transpose(rows[1], tbuf[1], iota16)
    _fire_out(out_hbm, tbuf[1], osem[1], b0, 1)
    _fire_gather(table_hbm, idx_v, rows[1], gsem[1], 3)

    def slot(t, b):
        _wait_out(out_hbm, tbuf[b], osem[b])
        _wait_rows(table_hbm, rows[b], gsem[b])
        _transpose(rows[b], tbuf[b], iota16)
        _fire_out(out_hbm, tbuf[b], osem[b], b0, t)
        _fire_gather(table_hbm, idx_v, rows[b], gsem[b], t + 2)

    def pair(g, carry):
        slot(2 + 2 * g, 0)
        slot(3 + 2 * g, 1)
        return carry

    lax.fori_loop(0, (T - 4) // 2, pair, 0)

    for t, b in ((T - 2, 0), (T - 1, 1)):
        _wait_out(out_hbm, tbuf[b], osem[b])
        _wait_rows(table_hbm, rows[b], gsem[b])
        _transpose(rows[b], tbuf[b], iota16)
        _fire_out(out_hbm, tbuf[b], osem[b], b0, t)
    _wait_out(out_hbm, tbuf[0], osem[0])
    _wait_out(out_hbm, tbuf[1], osem[1])


def _gather_sc(idx_t, table):
    mesh = plsc.VectorSubcoreMesh(core_axis_name="c", subcore_axis_name="s")
    kern = pl.kernel(
        _body,
        out_type=jax.ShapeDtypeStruct((T, EMBED_DIM, B), jnp.float32),
        mesh=mesh,
        scratch_types=[
            pltpu.VMEM((T, PER_B), jnp.int32),
            pltpu.VMEM((PER_B, EMBED_DIM), jnp.float32),
            pltpu.VMEM((PER_B, EMBED_DIM), jnp.float32),
            pltpu.VMEM((EMBED_DIM, PER_B), jnp.float32),
            pltpu.VMEM((EMBED_DIM, PER_B), jnp.float32),
            pltpu.SemaphoreType.DMA,
            pltpu.SemaphoreType.DMA,
            pltpu.SemaphoreType.DMA,
            pltpu.SemaphoreType.DMA,
        ],
        compiler_params=pltpu.CompilerParams(
            use_tc_tiling_on_sc=False, needs_layout_passes=False
        ),
    )
    return kern(idx_t, table)


@jax.jit
def _embed(idx_t, table):
    g = _gather_sc(idx_t, table)
    return jnp.transpose(g, (2, 0, 1))


def kernel(action_tokens, table):
    idx_t = action_tokens.T.astype(jnp.int32)
    return _embed(idx_t, table)
